# baseline (speedup 1.0000x reference)
import functools

import jax
import jax.numpy as jnp
from jax import lax
from jax.experimental import pallas as pl
from jax.experimental.pallas import tpu as pltpu
from jax.experimental.pallas import tpu_sc as plsc

N = 10000
H = 128
E = 320000
G = 64
NCLS = 16

NC = 2
NS = 16
CH = 128

NPAD = 10240
SL = NPAD // NS
NROWS = 10112
RT = NROWS // NS
DUMMY_R = N + 64
DUMMY_S = N + 64

EPT = E // (NC * NS)
CHP = 64
NCHT = 160
NBUF = 4
EPT_S = E // NS
NCHT_S = 160

_mesh = plsc.VectorSubcoreMesh(
    core_axis_name="c", subcore_axis_name="s", num_cores=NC, num_subcores=NS
)


def _fill_i32(ref, start, nvec, value):
    def body(i, _):
        ref[pl.ds(start + i * 16, 16)] = jnp.full((16,), value, jnp.int32)
        return 0
    lax.fori_loop(0, nvec, body, 0)


def _zero_vec_loop(ref, nvec):
    def body(i, _):
        ref[pl.ds(i * 16, 16)] = jnp.zeros((16,), ref.dtype)
        return 0
    lax.fori_loop(0, nvec, body, 0)


def _repack_2d(src1d, dst2d, nvec):
    nv_row = CH // 16

    def body(i, _):
        v = src1d[pl.ds(i * 16, 16)]
        dst2d[i // nv_row, pl.ds((i % nv_row) * 16, 16)] = v
        return 0
    lax.fori_loop(0, nvec, body, 0)


@functools.partial(
    pl.kernel,
    out_type=(
        jax.ShapeDtypeStruct((NPAD,), jnp.float32),
        jax.ShapeDtypeStruct((NPAD,), jnp.float32),
    ),
    mesh=_mesh,
    scratch_types=[
        pltpu.VMEM_SHARED((NPAD,), jnp.float32),
        pltpu.VMEM_SHARED((NPAD,), jnp.float32),
        pltpu.VMEM_SHARED((NPAD,), jnp.float32),
        pltpu.VMEM((NCHT_S * CH,), jnp.int32),
        pltpu.VMEM((NCHT_S * CH,), jnp.int32),
        pltpu.VMEM((NCHT_S, CH), jnp.int32),
        pltpu.VMEM((NCHT_S * CH,), jnp.float32),
        pltpu.VMEM((CH,), jnp.float32),
        pltpu.VMEM((NPAD,), jnp.float32),
        pltpu.VMEM((SL,), jnp.float32),
        pltpu.VMEM((SL,), jnp.float32),
        pltpu.SemaphoreType.DMA,
    ],
    compiler_params=pltpu.CompilerParams(needs_layout_passes=False),
)
def _scalar_stage(src_hbm, dst_hbm, dinv_out, c_out,
                  deg_sh, g_sh, dinv_sh,
                  src1d, dst1d, dst2d, y1d, ones_v, dinv_loc, slv, slv2,
                  sem):
    s = lax.axis_index("s")
    c = lax.axis_index("c")
    base = s * SL

    _zero_vec_loop(slv, SL // 16)
    pltpu.sync_copy(slv, deg_sh.at[pl.ds(base, SL)])
    pltpu.sync_copy(slv, g_sh.at[pl.ds(base, SL)])

    def ones_body(i, _):
        ones_v[pl.ds(i * 16, 16)] = jnp.ones((16,), jnp.float32)
        return 0
    lax.fori_loop(0, CH // 16, ones_body, 0)

    ebase = s * EPT_S
    pltpu.sync_copy(src_hbm.at[pl.ds(ebase, EPT_S)], src1d.at[pl.ds(0, EPT_S)])
    pltpu.sync_copy(dst_hbm.at[pl.ds(ebase, EPT_S)], dst1d.at[pl.ds(0, EPT_S)])
    npadv = (NCHT_S * CH - EPT_S) // 16
    _fill_i32(src1d, EPT_S, npadv, 0)
    _fill_i32(dst1d, EPT_S, npadv, DUMMY_S)
    _repack_2d(dst1d, dst2d, NCHT_S * CH // 16)
    plsc.subcore_barrier()

    def deg_group(g2, _):
        for k in range(8):
            t = g2 * 8 + k
            pltpu.async_copy(ones_v, deg_sh.at[dst2d.at[t]], sem, add=True)
        for k in range(8):
            t = g2 * 8 + k
            pltpu.make_async_copy(ones_v, deg_sh.at[dst2d.at[t]], sem).wait()
        return 0
    lax.fori_loop(0, NCHT_S // 8, deg_group, 0)
    plsc.subcore_barrier()

    pltpu.sync_copy(deg_sh.at[pl.ds(base, SL)], slv)

    def rs_body(i, _):
        x = slv[pl.ds(i * 16, 16)] + 1.0
        bits = lax.bitcast_convert_type(x, jnp.int32)
        y0 = lax.bitcast_convert_type(
            jnp.full((16,), 0x5F3759DF, jnp.int32)
            - lax.shift_right_logical(bits, 1),
            jnp.float32,
        )
        y = y0
        for _u in range(3):
            y = y * (1.5 - 0.5 * x * y * y)
        slv2[pl.ds(i * 16, 16)] = y
        return 0
    lax.fori_loop(0, SL // 16, rs_body, 0)
    pltpu.sync_copy(slv2, dinv_sh.at[pl.ds(base, SL)])

    @pl.when(c == 0)
    def _():
        pltpu.sync_copy(slv2, dinv_out.at[pl.ds(base, SL)])
    plsc.subcore_barrier()

    pltpu.sync_copy(dinv_sh, dinv_loc)

    def gv_body(i, _):
        idx = src1d[pl.ds(i * 16, 16)]
        y1d[pl.ds(i * 16, 16)] = plsc.load_gather(dinv_loc, [idx])
        return 0
    lax.fori_loop(0, NCHT_S * CH // 16, gv_body, 0)

    def g_group(g2, _):
        for k in range(8):
            t = g2 * 8 + k
            pltpu.async_copy(y1d.at[pl.ds(t * CH, CH)],
                             g_sh.at[dst2d.at[t]], sem, add=True)
        for k in range(8):
            t = g2 * 8 + k
            pltpu.make_async_copy(y1d.at[pl.ds(t * CH, CH)],
                                  g_sh.at[dst2d.at[t]], sem).wait()
        return 0
    lax.fori_loop(0, NCHT_S // 8, g_group, 0)
    plsc.subcore_barrier()

    pltpu.sync_copy(g_sh.at[pl.ds(base, SL)], slv)

    def c_body(i, _):
        d = slv2[pl.ds(i * 16, 16)]
        slv[pl.ds(i * 16, 16)] = d * (slv[pl.ds(i * 16, 16)] + d)
        return 0
    lax.fori_loop(0, SL // 16, c_body, 0)

    @pl.when(c == 0)
    def _():
        pltpu.sync_copy(slv, c_out.at[pl.ds(base, SL)])


NFULL = EPT // CHP
NSUB = CHP // 16


@functools.partial(
    pl.kernel,
    out_type=jax.ShapeDtypeStruct((NC, NROWS, H), jnp.float32),
    mesh=_mesh,
    scratch_types=(
        [pltpu.VMEM_SHARED((NROWS, H), jnp.float32)]
        + [pltpu.VMEM((CHP,), jnp.int32) for _ in range(2)]
        + [pltpu.VMEM((CHP,), jnp.int32) for _ in range(2)]
        + [pltpu.VMEM((CHP, H // 2), jnp.int32) for _ in range(2)]
        + [pltpu.VMEM((CHP, H), jnp.float32) for _ in range(2)]
        + [pltpu.SemaphoreType.DMA for _ in range(6)]
    ),
    compiler_params=pltpu.CompilerParams(needs_layout_passes=False,
                                         use_tc_tiling_on_sc=False),
)
def _prop_stage(y_hbm, src_hbm, dst_hbm, spart, acc_sh, *bufs):
    ibs = bufs[0:2]
    dbs = bufs[2:4]
    rows_bf = bufs[4:6]
    rows = bufs[6:8]
    isems = bufs[8:10]
    gsems = bufs[10:12]
    ssems = bufs[12:14]
    s = lax.axis_index("s")
    c = lax.axis_index("c")
    base_r = s * RT
    ebase = (c * NS + s) * EPT
    rows0 = rows[0]

    def zb_body(i, _):
        r = i // (H // 16)
        col = (i % (H // 16)) * 16
        rows0[r, pl.ds(col, 16)] = jnp.zeros((16,), jnp.float32)
        return 0
    lax.fori_loop(0, CHP * (H // 16), zb_body, 0)
    for k in range(RT // CHP):
        pltpu.sync_copy(rows0, acc_sh.at[pl.ds(base_r + k * CHP, CHP)])
    rem = RT % CHP
    if rem:
        pltpu.sync_copy(rows0.at[pl.ds(0, rem)],
                        acc_sh.at[pl.ds(base_r + (RT // CHP) * CHP, rem)])
    plsc.subcore_barrier()

    def idx_start(t, b):
        off = ebase + jnp.minimum(t * CHP, EPT - CHP)
        pltpu.async_copy(src_hbm.at[pl.ds(off, CHP)], ibs[b], isems[b])
        pltpu.async_copy(dst_hbm.at[pl.ds(off, CHP)], dbs[b], isems[b])

    def idx_wait(t, b):
        pltpu.make_async_copy(src_hbm.at[pl.ds(0, CHP)], ibs[b],
                              isems[b]).wait()
        pltpu.make_async_copy(dst_hbm.at[pl.ds(0, CHP)], dbs[b],
                              isems[b]).wait()

    def gather_start(t, b):
        pltpu.async_copy(y_hbm.at[ibs[b]], rows_bf[b], gsems[b])

    def gather_wait(t, b):
        pltpu.make_async_copy(y_hbm.at[ibs[b]], rows_bf[b], gsems[b]).wait()

    def convert(b):
        def cv(r, _):
            for cc in range(H // 32):
                v = rows_bf[b][r, pl.ds(cc * 16, 16)]
                lo = lax.bitcast_convert_type(
                    lax.shift_left(v, 16), jnp.float32)
                hi = lax.bitcast_convert_type(
                    lax.bitwise_and(v, jnp.int32(-65536)), jnp.float32)
                rows[b][r, pl.ds(cc * 32, 16)] = lo
                rows[b][r, pl.ds(cc * 32 + 16, 16)] = hi
            return 0
        lax.fori_loop(0, CHP, cv, 0)

    def _sub_scatter(b, k, start):
        idx = dbs[b][pl.ds(k * 16, 16)]
        src = rows[b].at[pl.ds(k * 16, 16)]
        if start:
            pltpu.async_copy(src, acc_sh.at[idx], ssems[b], add=True)
        else:
            pltpu.make_async_copy(src, acc_sh.at[idx], ssems[b]).wait()

    def _scatter(t, b, start):
        for k in range(NSUB):
            lim = NFULL if k < NSUB - 1 else NFULL + 1
            if isinstance(t, int):
                if t < lim:
                    _sub_scatter(b, k, start)
            else:
                @pl.when(t < lim)
                def _(k=k):
                    _sub_scatter(b, k, start)

    def scatter_start(t, b):
        _scatter(t, b, True)

    def scatter_wait(t, b):
        _scatter(t, b, False)

    idx_start(0, 0)
    idx_start(1, 1)
    idx_wait(0, 0)
    gather_start(0, 0)

    def step(t2, _):
        for bb in range(2):
            t = t2 * 2 + bb
            gather_wait(t, bb)

            @pl.when(t >= 2)
            def _():
                scatter_wait(t - 2, bb)

            @pl.when(t + 1 < NCHT)
            def _():
                idx_wait(t + 1, bb ^ 1)
                gather_start(t + 1, bb ^ 1)
            convert(bb)
            scatter_start(t, bb)

            @pl.when(t + 2 < NCHT)
            def _():
                idx_start(t + 2, bb)
        return 0
    lax.fori_loop(0, NCHT // 2, step, 0)
    scatter_wait(NCHT - 2, 0)
    scatter_wait(NCHT - 1, 1)
    plsc.subcore_barrier()

    pltpu.sync_copy(acc_sh.at[pl.ds(base_r, RT)],
                    spart.at[c].at[pl.ds(base_r, RT)])


BLK = 1000
NBLK = N // BLK


def _layer_a_body(c_ref, w0_ref, b0_ref, w1_ref, dinv_ref, o_ref):
    x1 = jnp.maximum(c_ref[...] * w0_ref[...] + b0_ref[...], 0.0)
    o_ref[...] = dinv_ref[...] * jnp.dot(
        x1, w1_ref[...], preferred_element_type=jnp.float32)


def _layer_a_call(c2d, W0, b0, W1, dinv2d):
    return pl.pallas_call(
        _layer_a_body,
        grid=(NBLK,),
        in_specs=[
            pl.BlockSpec((BLK, 1), lambda i: (i, 0)),
            pl.BlockSpec((1, H), lambda i: (0, 0)),
            pl.BlockSpec((1, H), lambda i: (0, 0)),
            pl.BlockSpec((H, H), lambda i: (0, 0)),
            pl.BlockSpec((BLK, 1), lambda i: (i, 0)),
        ],
        out_specs=pl.BlockSpec((BLK, H), lambda i: (i, 0)),
        out_shape=jax.ShapeDtypeStruct((N, H), jnp.float32),
    )(c2d, W0, b0, W1, dinv2d)


def _layer_b_body(sp_ref, y_ref, dinv_ref, b_ref, w_ref, o_ref):
    ssum = sp_ref[0] + sp_ref[1]
    x = jnp.maximum(dinv_ref[...] * (ssum + y_ref[...]) + b_ref[...], 0.0)
    o_ref[...] = dinv_ref[...] * jnp.dot(
        x, w_ref[...], preferred_element_type=jnp.float32)


def _layer_b_call(spart, y, dinv2d, b, W):
    return pl.pallas_call(
        _layer_b_body,
        grid=(NBLK,),
        in_specs=[
            pl.BlockSpec((NC, BLK, H), lambda i: (0, i, 0)),
            pl.BlockSpec((BLK, H), lambda i: (i, 0)),
            pl.BlockSpec((BLK, 1), lambda i: (i, 0)),
            pl.BlockSpec((1, H), lambda i: (0, 0)),
            pl.BlockSpec((H, H), lambda i: (0, 0)),
        ],
        out_specs=pl.BlockSpec((BLK, H), lambda i: (i, 0)),
        out_shape=jax.ShapeDtypeStruct((N, H), jnp.float32),
    )(spart, y, dinv2d, b, W)


def _final_body(sp_ref, y_ref, dinv_ref, b_ref, batch_ref, wp_ref, bp_ref,
                o_ref, acc, cnt):
    i = pl.program_id(0)

    @pl.when(i == 0)
    def _():
        acc[...] = jnp.zeros_like(acc)
        cnt[...] = jnp.zeros_like(cnt)

    ssum = sp_ref[0] + sp_ref[1]
    x = jnp.maximum(dinv_ref[...] * (ssum + y_ref[...]) + b_ref[...], 0.0)
    oh = (batch_ref[...] == lax.broadcasted_iota(jnp.int32, (BLK, G), 1)
          ).astype(jnp.float32)
    acc[...] += lax.dot_general(
        oh, x, (((0,), (0,)), ((), ())), preferred_element_type=jnp.float32)
    cnt[...] += lax.dot_general(
        oh, jnp.ones((BLK, 1), jnp.float32), (((0,), (0,)), ((), ())),
        preferred_element_type=jnp.float32)

    @pl.when(i == pl.num_programs(0) - 1)
    def _():
        pooled = acc[...] / jnp.maximum(cnt[...], 1.0)
        logits = jnp.dot(pooled, wp_ref[...],
                         preferred_element_type=jnp.float32) + bp_ref[...]
        m = jnp.max(logits, axis=1, keepdims=True)
        ex = jnp.exp(logits - m)
        o_ref[...] = logits - m - jnp.log(jnp.sum(ex, axis=1, keepdims=True))


def _final_call(spart, y, dinv2d, b, batch2d, Wp, bp):
    return pl.pallas_call(
        _final_body,
        grid=(NBLK,),
        in_specs=[
            pl.BlockSpec((NC, BLK, H), lambda i: (0, i, 0)),
            pl.BlockSpec((BLK, H), lambda i: (i, 0)),
            pl.BlockSpec((BLK, 1), lambda i: (i, 0)),
            pl.BlockSpec((1, H), lambda i: (0, 0)),
            pl.BlockSpec((BLK, 1), lambda i: (i, 0)),
            pl.BlockSpec((H, NCLS), lambda i: (0, 0)),
            pl.BlockSpec((1, NCLS), lambda i: (0, 0)),
        ],
        out_specs=pl.BlockSpec((G, NCLS), lambda i: (0, 0)),
        out_shape=jax.ShapeDtypeStruct((G, NCLS), jnp.float32),
        scratch_shapes=[
            pltpu.VMEM((G, H), jnp.float32),
            pltpu.VMEM((G, 1), jnp.float32),
        ],
    )(spart, y, dinv2d, b, batch2d, Wp, bp)


def _swiz(y):
    y_sw = (y.reshape(N, H // 32, 2, 16).swapaxes(2, 3).reshape(N, H)
            .astype(jnp.bfloat16))
    return lax.bitcast_convert_type(y_sw.reshape(N, H // 2, 2), jnp.int32)


def kernel(edge_index, batch, W0, b0, W1, b1, W2, b2, Wp, bp):
    src = edge_index[0]
    dst = edge_index[1]

    dinv_p, c_p = _scalar_stage(src, dst)
    dinv2d = dinv_p[:N, None]
    c2d = c_p[:N, None]

    y1 = _layer_a_call(c2d, W0, b0[None, :], W1, dinv2d)
    s1 = _prop_stage(_swiz(y1), src, dst)
    y2 = _layer_b_call(s1, y1, dinv2d, b1[None, :], W2)
    s2 = _prop_stage(_swiz(y2), src, dst)
    return _final_call(s2, y2, dinv2d, b2[None, :], batch[:, None],
                       Wp, bp[None, :])


# speedup vs baseline: 1.8083x; 1.8083x over previous
import functools

import jax
import jax.numpy as jnp
from jax import lax
from jax.experimental import pallas as pl
from jax.experimental.pallas import tpu as pltpu
from jax.experimental.pallas import tpu_sc as plsc

N = 10000
H = 128
E = 320000
G = 64
NCLS = 16

NC = 2
NS = 16
CH = 128

NPAD = 10240
SL = NPAD // NS
NROWS = 10112
RT = NROWS // NS
DUMMY_R = N + 64
DUMMY_S = N + 64

EPT = E // (NC * NS)
CHP = 128
NCHT = 80
NBUF = 2
EPT_S = E // NS
NCHT_S = 160

_mesh = plsc.VectorSubcoreMesh(
    core_axis_name="c", subcore_axis_name="s", num_cores=NC, num_subcores=NS
)


def _fill_i32(ref, start, nvec, value):
    def body(i, _):
        ref[pl.ds(start + i * 16, 16)] = jnp.full((16,), value, jnp.int32)
        return 0
    lax.fori_loop(0, nvec, body, 0)


def _zero_vec_loop(ref, nvec):
    def body(i, _):
        ref[pl.ds(i * 16, 16)] = jnp.zeros((16,), ref.dtype)
        return 0
    lax.fori_loop(0, nvec, body, 0)


def _repack_2d(src1d, dst2d, nvec):
    nv_row = CH // 16

    def body(i, _):
        v = src1d[pl.ds(i * 16, 16)]
        dst2d[i // nv_row, pl.ds((i % nv_row) * 16, 16)] = v
        return 0
    lax.fori_loop(0, nvec, body, 0)


@functools.partial(
    pl.kernel,
    out_type=(
        jax.ShapeDtypeStruct((NPAD,), jnp.float32),
        jax.ShapeDtypeStruct((NPAD,), jnp.float32),
    ),
    mesh=_mesh,
    scratch_types=[
        pltpu.VMEM_SHARED((NPAD,), jnp.float32),
        pltpu.VMEM_SHARED((NPAD,), jnp.float32),
        pltpu.VMEM_SHARED((NPAD,), jnp.float32),
        pltpu.VMEM((NCHT_S * CH,), jnp.int32),
        pltpu.VMEM((NCHT_S * CH,), jnp.int32),
        pltpu.VMEM((NCHT_S, CH), jnp.int32),
        pltpu.VMEM((NCHT_S * CH,), jnp.float32),
        pltpu.VMEM((CH,), jnp.float32),
        pltpu.VMEM((NPAD,), jnp.float32),
        pltpu.VMEM((SL,), jnp.float32),
        pltpu.VMEM((SL,), jnp.float32),
        pltpu.SemaphoreType.DMA,
    ],
    compiler_params=pltpu.CompilerParams(needs_layout_passes=False),
)
def _scalar_stage(src_hbm, dst_hbm, dinv_out, c_out,
                  deg_sh, g_sh, dinv_sh,
                  src1d, dst1d, dst2d, y1d, ones_v, dinv_loc, slv, slv2,
                  sem):
    s = lax.axis_index("s")
    c = lax.axis_index("c")
    base = s * SL

    _zero_vec_loop(slv, SL // 16)
    pltpu.sync_copy(slv, deg_sh.at[pl.ds(base, SL)])
    pltpu.sync_copy(slv, g_sh.at[pl.ds(base, SL)])

    def ones_body(i, _):
        ones_v[pl.ds(i * 16, 16)] = jnp.ones((16,), jnp.float32)
        return 0
    lax.fori_loop(0, CH // 16, ones_body, 0)

    ebase = s * EPT_S
    pltpu.sync_copy(src_hbm.at[pl.ds(ebase, EPT_S)], src1d.at[pl.ds(0, EPT_S)])
    pltpu.sync_copy(dst_hbm.at[pl.ds(ebase, EPT_S)], dst1d.at[pl.ds(0, EPT_S)])
    npadv = (NCHT_S * CH - EPT_S) // 16
    _fill_i32(src1d, EPT_S, npadv, 0)
    _fill_i32(dst1d, EPT_S, npadv, DUMMY_S)
    _repack_2d(dst1d, dst2d, NCHT_S * CH // 16)
    plsc.subcore_barrier()

    def deg_group(g2, _):
        for k in range(8):
            t = g2 * 8 + k
            pltpu.async_copy(ones_v, deg_sh.at[dst2d.at[t]], sem, add=True)
        for k in range(8):
            t = g2 * 8 + k
            pltpu.make_async_copy(ones_v, deg_sh.at[dst2d.at[t]], sem).wait()
        return 0
    lax.fori_loop(0, NCHT_S // 8, deg_group, 0)
    plsc.subcore_barrier()

    pltpu.sync_copy(deg_sh.at[pl.ds(base, SL)], slv)

    def rs_body(i, _):
        x = slv[pl.ds(i * 16, 16)] + 1.0
        bits = lax.bitcast_convert_type(x, jnp.int32)
        y0 = lax.bitcast_convert_type(
            jnp.full((16,), 0x5F3759DF, jnp.int32)
            - lax.shift_right_logical(bits, 1),
            jnp.float32,
        )
        y = y0
        for _u in range(3):
            y = y * (1.5 - 0.5 * x * y * y)
        slv2[pl.ds(i * 16, 16)] = y
        return 0
    lax.fori_loop(0, SL // 16, rs_body, 0)
    pltpu.sync_copy(slv2, dinv_sh.at[pl.ds(base, SL)])

    @pl.when(c == 0)
    def _():
        pltpu.sync_copy(slv2, dinv_out.at[pl.ds(base, SL)])
    plsc.subcore_barrier()

    pltpu.sync_copy(dinv_sh, dinv_loc)

    def gv_body(i, _):
        idx = src1d[pl.ds(i * 16, 16)]
        y1d[pl.ds(i * 16, 16)] = plsc.load_gather(dinv_loc, [idx])
        return 0
    lax.fori_loop(0, NCHT_S * CH // 16, gv_body, 0)

    def g_group(g2, _):
        for k in range(8):
            t = g2 * 8 + k
            pltpu.async_copy(y1d.at[pl.ds(t * CH, CH)],
                             g_sh.at[dst2d.at[t]], sem, add=True)
        for k in range(8):
            t = g2 * 8 + k
            pltpu.make_async_copy(y1d.at[pl.ds(t * CH, CH)],
                                  g_sh.at[dst2d.at[t]], sem).wait()
        return 0
    lax.fori_loop(0, NCHT_S // 8, g_group, 0)
    plsc.subcore_barrier()

    pltpu.sync_copy(g_sh.at[pl.ds(base, SL)], slv)

    def c_body(i, _):
        d = slv2[pl.ds(i * 16, 16)]
        slv[pl.ds(i * 16, 16)] = d * (slv[pl.ds(i * 16, 16)] + d)
        return 0
    lax.fori_loop(0, SL // 16, c_body, 0)

    @pl.when(c == 0)
    def _():
        pltpu.sync_copy(slv, c_out.at[pl.ds(base, SL)])


NFULL = EPT // CHP
NSUB = CHP // 16


@functools.partial(
    pl.kernel,
    out_type=jax.ShapeDtypeStruct((NC, NROWS, H), jnp.float32),
    mesh=_mesh,
    scratch_types=(
        [pltpu.VMEM_SHARED((NROWS, H), jnp.float32)]
        + [pltpu.VMEM((CHP,), jnp.int32) for _ in range(NBUF)]
        + [pltpu.VMEM((CHP,), jnp.int32) for _ in range(NBUF)]
        + [pltpu.VMEM((CHP, H), jnp.float32) for _ in range(NBUF)]
        + [pltpu.SemaphoreType.DMA for _ in range(3 * NBUF)]
    ),
    compiler_params=pltpu.CompilerParams(needs_layout_passes=False),
)
def _prop_stage(y_hbm, src_hbm, dst_hbm, spart, acc_sh, *bufs):
    ibs = bufs[:NBUF]
    dbs = bufs[NBUF:2 * NBUF]
    rows = bufs[2 * NBUF:3 * NBUF]
    isems = bufs[3 * NBUF:4 * NBUF]
    gsems = bufs[4 * NBUF:5 * NBUF]
    ssems = bufs[5 * NBUF:6 * NBUF]
    s = lax.axis_index("s")
    c = lax.axis_index("c")
    base_r = s * RT
    ebase = (c * NS + s) * EPT
    rows0 = rows[0]

    def zb_body(i, _):
        r = i // (H // 16)
        col = (i % (H // 16)) * 16
        rows0[r, pl.ds(col, 16)] = jnp.zeros((16,), jnp.float32)
        return 0
    lax.fori_loop(0, CHP * (H // 16), zb_body, 0)
    for k in range(RT // CHP):
        pltpu.sync_copy(rows0, acc_sh.at[pl.ds(base_r + k * CHP, CHP)])
    rem = RT % CHP
    if rem:
        pltpu.sync_copy(rows0.at[pl.ds(0, rem)],
                        acc_sh.at[pl.ds(base_r + (RT // CHP) * CHP, rem)])
    plsc.subcore_barrier()

    def idx_start(t, b):
        off = ebase + jnp.minimum(t * CHP, EPT - CHP)
        pltpu.async_copy(src_hbm.at[pl.ds(off, CHP)], ibs[b], isems[b])
        pltpu.async_copy(dst_hbm.at[pl.ds(off, CHP)], dbs[b], isems[b])

    def idx_wait(t, b):
        pltpu.make_async_copy(src_hbm.at[pl.ds(0, CHP)], ibs[b],
                              isems[b]).wait()
        pltpu.make_async_copy(dst_hbm.at[pl.ds(0, CHP)], dbs[b],
                              isems[b]).wait()

    def gather_start(t, b):
        pltpu.async_copy(y_hbm.at[ibs[b]], rows[b], gsems[b])

    def gather_wait(t, b):
        pltpu.make_async_copy(y_hbm.at[ibs[b]], rows[b], gsems[b]).wait()

    def _sub_scatter(b, k, start):
        idx = dbs[b][pl.ds(k * 16, 16)]
        src = rows[b].at[pl.ds(k * 16, 16)]
        if start:
            pltpu.async_copy(src, acc_sh.at[idx], ssems[b], add=True)
        else:
            pltpu.make_async_copy(src, acc_sh.at[idx], ssems[b]).wait()

    def _scatter(t, b, start):
        for k in range(NSUB):
            lim = NFULL if k < NSUB - 1 else NFULL + 1
            if isinstance(t, int):
                if t < lim:
                    _sub_scatter(b, k, start)
            else:
                @pl.when(t < lim)
                def _(k=k):
                    _sub_scatter(b, k, start)

    def scatter_start(t, b):
        _scatter(t, b, True)

    def scatter_wait(t, b):
        _scatter(t, b, False)

    idx_start(0, 0)

    def step(t2, _):
        for bb in range(2):
            t = t2 * 2 + bb

            @pl.when(t >= 2)
            def _():
                scatter_wait(t - 2, bb)

            @pl.when(t >= 1)
            def _():
                gather_wait(t - 1, bb ^ 1)
                scatter_start(t - 1, bb ^ 1)
            idx_wait(t, bb)
            gather_start(t, bb)

            @pl.when(t + 1 < NCHT)
            def _():
                idx_start(t + 1, bb ^ 1)
        return 0
    lax.fori_loop(0, NCHT // 2, step, 0)
    gather_wait(NCHT - 1, 1)
    scatter_start(NCHT - 1, 1)
    scatter_wait(NCHT - 2, 0)
    scatter_wait(NCHT - 1, 1)
    plsc.subcore_barrier()

    pltpu.sync_copy(acc_sh.at[pl.ds(base_r, RT)],
                    spart.at[c].at[pl.ds(base_r, RT)])


BLK = 1000
NBLK = N // BLK


def _layer_a_body(c_ref, w0_ref, b0_ref, w1_ref, dinv_ref, o_ref):
    x1 = jnp.maximum(c_ref[...] * w0_ref[...] + b0_ref[...], 0.0)
    o_ref[...] = dinv_ref[...] * jnp.dot(
        x1, w1_ref[...], preferred_element_type=jnp.float32)


def _layer_a_call(c2d, W0, b0, W1, dinv2d):
    return pl.pallas_call(
        _layer_a_body,
        grid=(NBLK,),
        in_specs=[
            pl.BlockSpec((BLK, 1), lambda i: (i, 0)),
            pl.BlockSpec((1, H), lambda i: (0, 0)),
            pl.BlockSpec((1, H), lambda i: (0, 0)),
            pl.BlockSpec((H, H), lambda i: (0, 0)),
            pl.BlockSpec((BLK, 1), lambda i: (i, 0)),
        ],
        out_specs=pl.BlockSpec((BLK, H), lambda i: (i, 0)),
        out_shape=jax.ShapeDtypeStruct((N, H), jnp.float32),
    )(c2d, W0, b0, W1, dinv2d)


def _layer_b_body(sp_ref, y_ref, dinv_ref, b_ref, w_ref, o_ref):
    ssum = sp_ref[0] + sp_ref[1]
    x = jnp.maximum(dinv_ref[...] * (ssum + y_ref[...]) + b_ref[...], 0.0)
    o_ref[...] = dinv_ref[...] * jnp.dot(
        x, w_ref[...], preferred_element_type=jnp.float32)


def _layer_b_call(spart, y, dinv2d, b, W):
    return pl.pallas_call(
        _layer_b_body,
        grid=(NBLK,),
        in_specs=[
            pl.BlockSpec((NC, BLK, H), lambda i: (0, i, 0)),
            pl.BlockSpec((BLK, H), lambda i: (i, 0)),
            pl.BlockSpec((BLK, 1), lambda i: (i, 0)),
            pl.BlockSpec((1, H), lambda i: (0, 0)),
            pl.BlockSpec((H, H), lambda i: (0, 0)),
        ],
        out_specs=pl.BlockSpec((BLK, H), lambda i: (i, 0)),
        out_shape=jax.ShapeDtypeStruct((N, H), jnp.float32),
    )(spart, y, dinv2d, b, W)


def _final_body(sp_ref, y_ref, dinv_ref, b_ref, batch_ref, wp_ref, bp_ref,
                o_ref, acc, cnt):
    i = pl.program_id(0)

    @pl.when(i == 0)
    def _():
        acc[...] = jnp.zeros_like(acc)
        cnt[...] = jnp.zeros_like(cnt)

    ssum = sp_ref[0] + sp_ref[1]
    x = jnp.maximum(dinv_ref[...] * (ssum + y_ref[...]) + b_ref[...], 0.0)
    oh = (batch_ref[...] == lax.broadcasted_iota(jnp.int32, (BLK, G), 1)
          ).astype(jnp.float32)
    acc[...] += lax.dot_general(
        oh, x, (((0,), (0,)), ((), ())), preferred_element_type=jnp.float32)
    cnt[...] += lax.dot_general(
        oh, jnp.ones((BLK, 1), jnp.float32), (((0,), (0,)), ((), ())),
        preferred_element_type=jnp.float32)

    @pl.when(i == pl.num_programs(0) - 1)
    def _():
        pooled = acc[...] / jnp.maximum(cnt[...], 1.0)
        logits = jnp.dot(pooled, wp_ref[...],
                         preferred_element_type=jnp.float32) + bp_ref[...]
        m = jnp.max(logits, axis=1, keepdims=True)
        ex = jnp.exp(logits - m)
        o_ref[...] = logits - m - jnp.log(jnp.sum(ex, axis=1, keepdims=True))


def _final_call(spart, y, dinv2d, b, batch2d, Wp, bp):
    return pl.pallas_call(
        _final_body,
        grid=(NBLK,),
        in_specs=[
            pl.BlockSpec((NC, BLK, H), lambda i: (0, i, 0)),
            pl.BlockSpec((BLK, H), lambda i: (i, 0)),
            pl.BlockSpec((BLK, 1), lambda i: (i, 0)),
            pl.BlockSpec((1, H), lambda i: (0, 0)),
            pl.BlockSpec((BLK, 1), lambda i: (i, 0)),
            pl.BlockSpec((H, NCLS), lambda i: (0, 0)),
            pl.BlockSpec((1, NCLS), lambda i: (0, 0)),
        ],
        out_specs=pl.BlockSpec((G, NCLS), lambda i: (0, 0)),
        out_shape=jax.ShapeDtypeStruct((G, NCLS), jnp.float32),
        scratch_shapes=[
            pltpu.VMEM((G, H), jnp.float32),
            pltpu.VMEM((G, 1), jnp.float32),
        ],
    )(spart, y, dinv2d, b, batch2d, Wp, bp)


def kernel(edge_index, batch, W0, b0, W1, b1, W2, b2, Wp, bp):
    src = edge_index[0]
    dst = edge_index[1]

    dinv_p, c_p = _scalar_stage(src, dst)
    dinv2d = dinv_p[:N, None]
    c2d = c_p[:N, None]

    y1 = _layer_a_call(c2d, W0, b0[None, :], W1, dinv2d)
    s1 = _prop_stage(y1, src, dst)
    y2 = _layer_b_call(s1, y1, dinv2d, b1[None, :], W2)
    s2 = _prop_stage(y2, src, dst)
    return _final_call(s2, y2, dinv2d, b2[None, :], batch[:, None],
                       Wp, bp[None, :])


# speedup vs baseline: 1.9252x; 1.0647x over previous
import functools

import jax
import jax.numpy as jnp
from jax import lax
from jax.experimental import pallas as pl
from jax.experimental.pallas import tpu as pltpu
from jax.experimental.pallas import tpu_sc as plsc

N = 10000
H = 128
E = 320000
G = 64
NCLS = 16

NC = 2
NS = 16
CH = 128

NPAD = 10240
SL = NPAD // NS
NROWS = 10112
RT = NROWS // NS
DUMMY_R = N + 64
DUMMY_S = N + 64

EPT = E // (NC * NS)
CHP = 64
NCHT = 160
NBUF = 4
EPT_S = E // NS
NCHT_S = 160

_mesh = plsc.VectorSubcoreMesh(
    core_axis_name="c", subcore_axis_name="s", num_cores=NC, num_subcores=NS
)


def _fill_i32(ref, start, nvec, value):
    def body(i, _):
        ref[pl.ds(start + i * 16, 16)] = jnp.full((16,), value, jnp.int32)
        return 0
    lax.fori_loop(0, nvec, body, 0)


def _zero_vec_loop(ref, nvec):
    def body(i, _):
        ref[pl.ds(i * 16, 16)] = jnp.zeros((16,), ref.dtype)
        return 0
    lax.fori_loop(0, nvec, body, 0)


def _repack_2d(src1d, dst2d, nvec):
    nv_row = CH // 16

    def body(i, _):
        v = src1d[pl.ds(i * 16, 16)]
        dst2d[i // nv_row, pl.ds((i % nv_row) * 16, 16)] = v
        return 0
    lax.fori_loop(0, nvec, body, 0)


@functools.partial(
    pl.kernel,
    out_type=(
        jax.ShapeDtypeStruct((NPAD,), jnp.float32),
        jax.ShapeDtypeStruct((NPAD,), jnp.float32),
    ),
    mesh=_mesh,
    scratch_types=[
        pltpu.VMEM_SHARED((NPAD,), jnp.float32),
        pltpu.VMEM_SHARED((NPAD,), jnp.float32),
        pltpu.VMEM_SHARED((NPAD,), jnp.float32),
        pltpu.VMEM((NCHT_S * CH,), jnp.int32),
        pltpu.VMEM((NCHT_S * CH,), jnp.int32),
        pltpu.VMEM((NCHT_S, CH), jnp.int32),
        pltpu.VMEM((NCHT_S * CH,), jnp.float32),
        pltpu.VMEM((CH,), jnp.float32),
        pltpu.VMEM((NPAD,), jnp.float32),
        pltpu.VMEM((SL,), jnp.float32),
        pltpu.VMEM((SL,), jnp.float32),
        pltpu.SemaphoreType.DMA,
    ],
    compiler_params=pltpu.CompilerParams(needs_layout_passes=False),
)
def _scalar_stage(src_hbm, dst_hbm, dinv_out, c_out,
                  deg_sh, g_sh, dinv_sh,
                  src1d, dst1d, dst2d, y1d, ones_v, dinv_loc, slv, slv2,
                  sem):
    s = lax.axis_index("s")
    c = lax.axis_index("c")
    base = s * SL

    _zero_vec_loop(slv, SL // 16)
    pltpu.sync_copy(slv, deg_sh.at[pl.ds(base, SL)])
    pltpu.sync_copy(slv, g_sh.at[pl.ds(base, SL)])

    def ones_body(i, _):
        ones_v[pl.ds(i * 16, 16)] = jnp.ones((16,), jnp.float32)
        return 0
    lax.fori_loop(0, CH // 16, ones_body, 0)

    ebase = s * EPT_S
    pltpu.sync_copy(src_hbm.at[pl.ds(ebase, EPT_S)], src1d.at[pl.ds(0, EPT_S)])
    pltpu.sync_copy(dst_hbm.at[pl.ds(ebase, EPT_S)], dst1d.at[pl.ds(0, EPT_S)])
    npadv = (NCHT_S * CH - EPT_S) // 16
    _fill_i32(src1d, EPT_S, npadv, 0)
    _fill_i32(dst1d, EPT_S, npadv, DUMMY_S)
    _repack_2d(dst1d, dst2d, NCHT_S * CH // 16)
    plsc.subcore_barrier()

    def deg_group(g2, _):
        for k in range(16):
            t = g2 * 16 + k
            pltpu.async_copy(ones_v, deg_sh.at[dst2d.at[t]], sem, add=True)
        for k in range(16):
            t = g2 * 16 + k
            pltpu.make_async_copy(ones_v, deg_sh.at[dst2d.at[t]], sem).wait()
        return 0
    lax.fori_loop(0, NCHT_S // 16, deg_group, 0)
    plsc.subcore_barrier()

    pltpu.sync_copy(deg_sh.at[pl.ds(base, SL)], slv)

    def rs_body(i, _):
        x = slv[pl.ds(i * 16, 16)] + 1.0
        bits = lax.bitcast_convert_type(x, jnp.int32)
        y0 = lax.bitcast_convert_type(
            jnp.full((16,), 0x5F3759DF, jnp.int32)
            - lax.shift_right_logical(bits, 1),
            jnp.float32,
        )
        y = y0
        for _u in range(3):
            y = y * (1.5 - 0.5 * x * y * y)
        slv2[pl.ds(i * 16, 16)] = y
        return 0
    lax.fori_loop(0, SL // 16, rs_body, 0)
    pltpu.sync_copy(slv2, dinv_sh.at[pl.ds(base, SL)])

    @pl.when(c == 0)
    def _():
        pltpu.sync_copy(slv2, dinv_out.at[pl.ds(base, SL)])
    plsc.subcore_barrier()

    pltpu.sync_copy(dinv_sh, dinv_loc)

    def gv_body(i, _):
        idx = src1d[pl.ds(i * 16, 16)]
        y1d[pl.ds(i * 16, 16)] = plsc.load_gather(dinv_loc, [idx])
        return 0
    lax.fori_loop(0, NCHT_S * CH // 16, gv_body, 0)

    def g_group(g2, _):
        for k in range(16):
            t = g2 * 16 + k
            pltpu.async_copy(y1d.at[pl.ds(t * CH, CH)],
                             g_sh.at[dst2d.at[t]], sem, add=True)
        for k in range(16):
            t = g2 * 16 + k
            pltpu.make_async_copy(y1d.at[pl.ds(t * CH, CH)],
                                  g_sh.at[dst2d.at[t]], sem).wait()
        return 0
    lax.fori_loop(0, NCHT_S // 16, g_group, 0)
    plsc.subcore_barrier()

    pltpu.sync_copy(g_sh.at[pl.ds(base, SL)], slv)

    def c_body(i, _):
        d = slv2[pl.ds(i * 16, 16)]
        slv[pl.ds(i * 16, 16)] = d * (slv[pl.ds(i * 16, 16)] + d)
        return 0
    lax.fori_loop(0, SL // 16, c_body, 0)

    @pl.when(c == 0)
    def _():
        pltpu.sync_copy(slv, c_out.at[pl.ds(base, SL)])


NFULL = EPT // CHP
NSUB = CHP // 16


@functools.partial(
    pl.kernel,
    out_type=jax.ShapeDtypeStruct((NC, NROWS, H), jnp.float32),
    mesh=_mesh,
    scratch_types=(
        [pltpu.VMEM_SHARED((NROWS, H), jnp.float32)]
        + [pltpu.VMEM((CHP,), jnp.int32) for _ in range(NBUF)]
        + [pltpu.VMEM((CHP,), jnp.int32) for _ in range(NBUF)]
        + [pltpu.VMEM((CHP, H), jnp.float32) for _ in range(NBUF)]
        + [pltpu.SemaphoreType.DMA for _ in range(3 * NBUF)]
    ),
    compiler_params=pltpu.CompilerParams(needs_layout_passes=False),
)
def _prop_stage(y_hbm, src_hbm, dst_hbm, spart, acc_sh, *bufs):
    ibs = bufs[:NBUF]
    dbs = bufs[NBUF:2 * NBUF]
    rows = bufs[2 * NBUF:3 * NBUF]
    isems = bufs[3 * NBUF:4 * NBUF]
    gsems = bufs[4 * NBUF:5 * NBUF]
    ssems = bufs[5 * NBUF:6 * NBUF]
    s = lax.axis_index("s")
    c = lax.axis_index("c")
    base_r = s * RT
    ebase = (c * NS + s) * EPT
    rows0 = rows[0]

    def zb_body(i, _):
        r = i // (H // 16)
        col = (i % (H // 16)) * 16
        rows0[r, pl.ds(col, 16)] = jnp.zeros((16,), jnp.float32)
        return 0
    lax.fori_loop(0, CHP * (H // 16), zb_body, 0)
    for k in range(RT // CHP):
        pltpu.sync_copy(rows0, acc_sh.at[pl.ds(base_r + k * CHP, CHP)])
    rem = RT % CHP
    if rem:
        pltpu.sync_copy(rows0.at[pl.ds(0, rem)],
                        acc_sh.at[pl.ds(base_r + (RT // CHP) * CHP, rem)])
    plsc.subcore_barrier()

    def idx_start(t, b):
        off = ebase + jnp.minimum(t * CHP, EPT - CHP)
        pltpu.async_copy(src_hbm.at[pl.ds(off, CHP)], ibs[b], isems[b])
        pltpu.async_copy(dst_hbm.at[pl.ds(off, CHP)], dbs[b], isems[b])

    def idx_wait(t, b):
        pltpu.make_async_copy(src_hbm.at[pl.ds(0, CHP)], ibs[b],
                              isems[b]).wait()
        pltpu.make_async_copy(dst_hbm.at[pl.ds(0, CHP)], dbs[b],
                              isems[b]).wait()

    def gather_start(t, b):
        pltpu.async_copy(y_hbm.at[ibs[b]], rows[b], gsems[b])

    def gather_wait(t, b):
        pltpu.make_async_copy(y_hbm.at[ibs[b]], rows[b], gsems[b]).wait()

    def _sub_scatter(b, k, start):
        idx = dbs[b][pl.ds(k * 16, 16)]
        src = rows[b].at[pl.ds(k * 16, 16)]
        if start:
            pltpu.async_copy(src, acc_sh.at[idx], ssems[b], add=True)
        else:
            pltpu.make_async_copy(src, acc_sh.at[idx], ssems[b]).wait()

    def _scatter(t, b, start):
        for k in range(NSUB):
            lim = NFULL if k < NSUB - 1 else NFULL + 1
            if isinstance(t, int):
                if t < lim:
                    _sub_scatter(b, k, start)
            else:
                @pl.when(t < lim)
                def _(k=k):
                    _sub_scatter(b, k, start)

    def scatter_start(t, b):
        _scatter(t, b, True)

    def scatter_wait(t, b):
        _scatter(t, b, False)

    idx_start(0, 0)
    idx_start(1, 1)

    def step(t4, _):
        for bb in range(NBUF):
            t = t4 * NBUF + bb
            b2 = (bb + 2) % NBUF

            @pl.when(t >= NBUF)
            def _():
                scatter_wait(t - NBUF, bb)

            @pl.when(t >= 2)
            def _():
                gather_wait(t - 2, b2)
                scatter_start(t - 2, b2)
            idx_wait(t, bb)
            gather_start(t, bb)

            @pl.when(t + 2 < NCHT)
            def _():
                idx_start(t + 2, b2)
        return 0
    lax.fori_loop(0, NCHT // NBUF, step, 0)
    gather_wait(NCHT - 2, (NCHT - 2) % NBUF)
    scatter_start(NCHT - 2, (NCHT - 2) % NBUF)
    gather_wait(NCHT - 1, (NCHT - 1) % NBUF)
    scatter_start(NCHT - 1, (NCHT - 1) % NBUF)
    for t in range(NCHT - NBUF, NCHT):
        scatter_wait(t, t % NBUF)
    plsc.subcore_barrier()

    pltpu.sync_copy(acc_sh.at[pl.ds(base_r, RT)],
                    spart.at[c].at[pl.ds(base_r, RT)])


BLK = 1000
NBLK = N // BLK


def _layer_a_body(c_ref, w0_ref, b0_ref, w1_ref, dinv_ref, o_ref):
    x1 = jnp.maximum(c_ref[...] * w0_ref[...] + b0_ref[...], 0.0)
    o_ref[...] = dinv_ref[...] * jnp.dot(
        x1, w1_ref[...], preferred_element_type=jnp.float32)


def _layer_a_call(c2d, W0, b0, W1, dinv2d):
    return pl.pallas_call(
        _layer_a_body,
        grid=(NBLK,),
        in_specs=[
            pl.BlockSpec((BLK, 1), lambda i: (i, 0)),
            pl.BlockSpec((1, H), lambda i: (0, 0)),
            pl.BlockSpec((1, H), lambda i: (0, 0)),
            pl.BlockSpec((H, H), lambda i: (0, 0)),
            pl.BlockSpec((BLK, 1), lambda i: (i, 0)),
        ],
        out_specs=pl.BlockSpec((BLK, H), lambda i: (i, 0)),
        out_shape=jax.ShapeDtypeStruct((N, H), jnp.float32),
    )(c2d, W0, b0, W1, dinv2d)


def _layer_b_body(sp_ref, y_ref, dinv_ref, b_ref, w_ref, o_ref):
    ssum = sp_ref[0] + sp_ref[1]
    x = jnp.maximum(dinv_ref[...] * (ssum + y_ref[...]) + b_ref[...], 0.0)
    o_ref[...] = dinv_ref[...] * jnp.dot(
        x, w_ref[...], preferred_element_type=jnp.float32)


def _layer_b_call(spart, y, dinv2d, b, W):
    return pl.pallas_call(
        _layer_b_body,
        grid=(NBLK,),
        in_specs=[
            pl.BlockSpec((NC, BLK, H), lambda i: (0, i, 0)),
            pl.BlockSpec((BLK, H), lambda i: (i, 0)),
            pl.BlockSpec((BLK, 1), lambda i: (i, 0)),
            pl.BlockSpec((1, H), lambda i: (0, 0)),
            pl.BlockSpec((H, H), lambda i: (0, 0)),
        ],
        out_specs=pl.BlockSpec((BLK, H), lambda i: (i, 0)),
        out_shape=jax.ShapeDtypeStruct((N, H), jnp.float32),
    )(spart, y, dinv2d, b, W)


def _final_body(sp_ref, y_ref, dinv_ref, b_ref, batch_ref, wp_ref, bp_ref,
                o_ref, acc, cnt):
    i = pl.program_id(0)

    @pl.when(i == 0)
    def _():
        acc[...] = jnp.zeros_like(acc)
        cnt[...] = jnp.zeros_like(cnt)

    ssum = sp_ref[0] + sp_ref[1]
    x = jnp.maximum(dinv_ref[...] * (ssum + y_ref[...]) + b_ref[...], 0.0)
    oh = (batch_ref[...] == lax.broadcasted_iota(jnp.int32, (BLK, G), 1)
          ).astype(jnp.float32)
    acc[...] += lax.dot_general(
        oh, x, (((0,), (0,)), ((), ())), preferred_element_type=jnp.float32)
    cnt[...] += lax.dot_general(
        oh, jnp.ones((BLK, 1), jnp.float32), (((0,), (0,)), ((), ())),
        preferred_element_type=jnp.float32)

    @pl.when(i == pl.num_programs(0) - 1)
    def _():
        pooled = acc[...] / jnp.maximum(cnt[...], 1.0)
        logits = jnp.dot(pooled, wp_ref[...],
                         preferred_element_type=jnp.float32) + bp_ref[...]
        m = jnp.max(logits, axis=1, keepdims=True)
        ex = jnp.exp(logits - m)
        o_ref[...] = logits - m - jnp.log(jnp.sum(ex, axis=1, keepdims=True))


def _final_call(spart, y, dinv2d, b, batch2d, Wp, bp):
    return pl.pallas_call(
        _final_body,
        grid=(NBLK,),
        in_specs=[
            pl.BlockSpec((NC, BLK, H), lambda i: (0, i, 0)),
            pl.BlockSpec((BLK, H), lambda i: (i, 0)),
            pl.BlockSpec((BLK, 1), lambda i: (i, 0)),
            pl.BlockSpec((1, H), lambda i: (0, 0)),
            pl.BlockSpec((BLK, 1), lambda i: (i, 0)),
            pl.BlockSpec((H, NCLS), lambda i: (0, 0)),
            pl.BlockSpec((1, NCLS), lambda i: (0, 0)),
        ],
        out_specs=pl.BlockSpec((G, NCLS), lambda i: (0, 0)),
        out_shape=jax.ShapeDtypeStruct((G, NCLS), jnp.float32),
        scratch_shapes=[
            pltpu.VMEM((G, H), jnp.float32),
            pltpu.VMEM((G, 1), jnp.float32),
        ],
    )(spart, y, dinv2d, b, batch2d, Wp, bp)


def kernel(edge_index, batch, W0, b0, W1, b1, W2, b2, Wp, bp):
    src = edge_index[0]
    dst = edge_index[1]

    dinv_p, c_p = _scalar_stage(src, dst)
    dinv2d = dinv_p[:N, None]
    c2d = c_p[:N, None]

    y1 = _layer_a_call(c2d, W0, b0[None, :], W1, dinv2d)
    s1 = _prop_stage(y1, src, dst)
    y2 = _layer_b_call(s1, y1, dinv2d, b1[None, :], W2)
    s2 = _prop_stage(y2, src, dst)
    return _final_call(s2, y2, dinv2d, b2[None, :], batch[:, None],
                       Wp, bp[None, :])


# speedup vs baseline: 1.9268x; 1.0008x over previous
import functools

import jax
import jax.numpy as jnp
from jax import lax
from jax.experimental import pallas as pl
from jax.experimental.pallas import tpu as pltpu
from jax.experimental.pallas import tpu_sc as plsc

N = 10000
H = 128
E = 320000
G = 64
NCLS = 16

NC = 2
NS = 16
CH = 128

NPAD = 10240
SL = NPAD // NS
NROWS = 10112
RT = NROWS // NS
DUMMY_R = N + 64
DUMMY_S = N + 64

EPT = E // (NC * NS)
CHP = 64
NCHT = 160
NBUF = 5
EPT_S = E // NS
NCHT_S = 160

_mesh = plsc.VectorSubcoreMesh(
    core_axis_name="c", subcore_axis_name="s", num_cores=NC, num_subcores=NS
)


def _fill_i32(ref, start, nvec, value):
    def body(i, _):
        ref[pl.ds(start + i * 16, 16)] = jnp.full((16,), value, jnp.int32)
        return 0
    lax.fori_loop(0, nvec, body, 0)


def _zero_vec_loop(ref, nvec):
    def body(i, _):
        ref[pl.ds(i * 16, 16)] = jnp.zeros((16,), ref.dtype)
        return 0
    lax.fori_loop(0, nvec, body, 0)


def _repack_2d(src1d, dst2d, nvec):
    nv_row = CH // 16

    def body(i, _):
        v = src1d[pl.ds(i * 16, 16)]
        dst2d[i // nv_row, pl.ds((i % nv_row) * 16, 16)] = v
        return 0
    lax.fori_loop(0, nvec, body, 0)


@functools.partial(
    pl.kernel,
    out_type=(
        jax.ShapeDtypeStruct((NPAD,), jnp.float32),
        jax.ShapeDtypeStruct((NPAD,), jnp.float32),
    ),
    mesh=_mesh,
    scratch_types=[
        pltpu.VMEM_SHARED((NPAD,), jnp.float32),
        pltpu.VMEM_SHARED((NPAD,), jnp.float32),
        pltpu.VMEM_SHARED((NPAD,), jnp.float32),
        pltpu.VMEM((NCHT_S * CH,), jnp.int32),
        pltpu.VMEM((NCHT_S * CH,), jnp.int32),
        pltpu.VMEM((NCHT_S, CH), jnp.int32),
        pltpu.VMEM((NCHT_S * CH,), jnp.float32),
        pltpu.VMEM((CH,), jnp.float32),
        pltpu.VMEM((NPAD,), jnp.float32),
        pltpu.VMEM((SL,), jnp.float32),
        pltpu.VMEM((SL,), jnp.float32),
        pltpu.SemaphoreType.DMA,
    ],
    compiler_params=pltpu.CompilerParams(needs_layout_passes=False),
)
def _scalar_stage(src_hbm, dst_hbm, dinv_out, c_out,
                  deg_sh, g_sh, dinv_sh,
                  src1d, dst1d, dst2d, y1d, ones_v, dinv_loc, slv, slv2,
                  sem):
    s = lax.axis_index("s")
    c = lax.axis_index("c")
    base = s * SL

    _zero_vec_loop(slv, SL // 16)
    pltpu.sync_copy(slv, deg_sh.at[pl.ds(base, SL)])
    pltpu.sync_copy(slv, g_sh.at[pl.ds(base, SL)])

    def ones_body(i, _):
        ones_v[pl.ds(i * 16, 16)] = jnp.ones((16,), jnp.float32)
        return 0
    lax.fori_loop(0, CH // 16, ones_body, 0)

    ebase = s * EPT_S
    pltpu.sync_copy(src_hbm.at[pl.ds(ebase, EPT_S)], src1d.at[pl.ds(0, EPT_S)])
    pltpu.sync_copy(dst_hbm.at[pl.ds(ebase, EPT_S)], dst1d.at[pl.ds(0, EPT_S)])
    npadv = (NCHT_S * CH - EPT_S) // 16
    _fill_i32(src1d, EPT_S, npadv, 0)
    _fill_i32(dst1d, EPT_S, npadv, DUMMY_S)
    _repack_2d(dst1d, dst2d, NCHT_S * CH // 16)
    plsc.subcore_barrier()

    def deg_group(g2, _):
        for k in range(16):
            t = g2 * 16 + k
            pltpu.async_copy(ones_v, deg_sh.at[dst2d.at[t]], sem, add=True)
        for k in range(16):
            t = g2 * 16 + k
            pltpu.make_async_copy(ones_v, deg_sh.at[dst2d.at[t]], sem).wait()
        return 0
    lax.fori_loop(0, NCHT_S // 16, deg_group, 0)
    plsc.subcore_barrier()

    pltpu.sync_copy(deg_sh.at[pl.ds(base, SL)], slv)

    def rs_body(i, _):
        x = slv[pl.ds(i * 16, 16)] + 1.0
        bits = lax.bitcast_convert_type(x, jnp.int32)
        y0 = lax.bitcast_convert_type(
            jnp.full((16,), 0x5F3759DF, jnp.int32)
            - lax.shift_right_logical(bits, 1),
            jnp.float32,
        )
        y = y0
        for _u in range(3):
            y = y * (1.5 - 0.5 * x * y * y)
        slv2[pl.ds(i * 16, 16)] = y
        return 0
    lax.fori_loop(0, SL // 16, rs_body, 0)
    pltpu.sync_copy(slv2, dinv_sh.at[pl.ds(base, SL)])

    @pl.when(c == 0)
    def _():
        pltpu.sync_copy(slv2, dinv_out.at[pl.ds(base, SL)])
    plsc.subcore_barrier()

    pltpu.sync_copy(dinv_sh, dinv_loc)

    def gv_body(i, _):
        idx = src1d[pl.ds(i * 16, 16)]
        y1d[pl.ds(i * 16, 16)] = plsc.load_gather(dinv_loc, [idx])
        return 0
    lax.fori_loop(0, NCHT_S * CH // 16, gv_body, 0)

    def g_group(g2, _):
        for k in range(16):
            t = g2 * 16 + k
            pltpu.async_copy(y1d.at[pl.ds(t * CH, CH)],
                             g_sh.at[dst2d.at[t]], sem, add=True)
        for k in range(16):
            t = g2 * 16 + k
            pltpu.make_async_copy(y1d.at[pl.ds(t * CH, CH)],
                                  g_sh.at[dst2d.at[t]], sem).wait()
        return 0
    lax.fori_loop(0, NCHT_S // 16, g_group, 0)
    plsc.subcore_barrier()

    pltpu.sync_copy(g_sh.at[pl.ds(base, SL)], slv)

    def c_body(i, _):
        d = slv2[pl.ds(i * 16, 16)]
        slv[pl.ds(i * 16, 16)] = d * (slv[pl.ds(i * 16, 16)] + d)
        return 0
    lax.fori_loop(0, SL // 16, c_body, 0)

    @pl.when(c == 0)
    def _():
        pltpu.sync_copy(slv, c_out.at[pl.ds(base, SL)])


NFULL = EPT // CHP
NSUB = CHP // 16


@functools.partial(
    pl.kernel,
    out_type=jax.ShapeDtypeStruct((NC, NROWS, H), jnp.float32),
    mesh=_mesh,
    scratch_types=(
        [pltpu.VMEM_SHARED((NROWS, H), jnp.float32)]
        + [pltpu.VMEM((CHP,), jnp.int32) for _ in range(NBUF)]
        + [pltpu.VMEM((CHP,), jnp.int32) for _ in range(NBUF)]
        + [pltpu.VMEM((CHP, H), jnp.float32) for _ in range(NBUF)]
        + [pltpu.SemaphoreType.DMA for _ in range(3 * NBUF)]
    ),
    compiler_params=pltpu.CompilerParams(needs_layout_passes=False),
)
def _prop_stage(y_hbm, src_hbm, dst_hbm, spart, acc_sh, *bufs):
    ibs = bufs[:NBUF]
    dbs = bufs[NBUF:2 * NBUF]
    rows = bufs[2 * NBUF:3 * NBUF]
    isems = bufs[3 * NBUF:4 * NBUF]
    gsems = bufs[4 * NBUF:5 * NBUF]
    ssems = bufs[5 * NBUF:6 * NBUF]
    s = lax.axis_index("s")
    c = lax.axis_index("c")
    base_r = s * RT
    ebase = (c * NS + s) * EPT
    rows0 = rows[0]

    def zb_body(i, _):
        r = i // (H // 16)
        col = (i % (H // 16)) * 16
        rows0[r, pl.ds(col, 16)] = jnp.zeros((16,), jnp.float32)
        return 0
    lax.fori_loop(0, CHP * (H // 16), zb_body, 0)
    for k in range(RT // CHP):
        pltpu.sync_copy(rows0, acc_sh.at[pl.ds(base_r + k * CHP, CHP)])
    rem = RT % CHP
    if rem:
        pltpu.sync_copy(rows0.at[pl.ds(0, rem)],
                        acc_sh.at[pl.ds(base_r + (RT // CHP) * CHP, rem)])
    plsc.subcore_barrier()

    def idx_start(t, b):
        off = ebase + jnp.minimum(t * CHP, EPT - CHP)
        pltpu.async_copy(src_hbm.at[pl.ds(off, CHP)], ibs[b], isems[b])
        pltpu.async_copy(dst_hbm.at[pl.ds(off, CHP)], dbs[b], isems[b])

    def idx_wait(t, b):
        pltpu.make_async_copy(src_hbm.at[pl.ds(0, CHP)], ibs[b],
                              isems[b]).wait()
        pltpu.make_async_copy(dst_hbm.at[pl.ds(0, CHP)], dbs[b],
                              isems[b]).wait()

    def gather_start(t, b):
        pltpu.async_copy(y_hbm.at[ibs[b]], rows[b], gsems[b])

    def gather_wait(t, b):
        pltpu.make_async_copy(y_hbm.at[ibs[b]], rows[b], gsems[b]).wait()

    def _sub_scatter(b, k, start):
        idx = dbs[b][pl.ds(k * 16, 16)]
        src = rows[b].at[pl.ds(k * 16, 16)]
        if start:
            pltpu.async_copy(src, acc_sh.at[idx], ssems[b], add=True)
        else:
            pltpu.make_async_copy(src, acc_sh.at[idx], ssems[b]).wait()

    def _scatter(t, b, start):
        for k in range(NSUB):
            lim = NFULL if k < NSUB - 1 else NFULL + 1
            if isinstance(t, int):
                if t < lim:
                    _sub_scatter(b, k, start)
            else:
                @pl.when(t < lim)
                def _(k=k):
                    _sub_scatter(b, k, start)

    def scatter_start(t, b):
        _scatter(t, b, True)

    def scatter_wait(t, b):
        _scatter(t, b, False)

    idx_start(0, 0)
    idx_start(1, 1)

    def step(t4, _):
        for bb in range(NBUF):
            t = t4 * NBUF + bb
            bm2 = (bb - 2) % NBUF
            bp2 = (bb + 2) % NBUF

            @pl.when(t >= NBUF)
            def _():
                scatter_wait(t - NBUF, bb)

            @pl.when(t >= 2)
            def _():
                gather_wait(t - 2, bm2)
                scatter_start(t - 2, bm2)
            idx_wait(t, bb)
            gather_start(t, bb)

            @pl.when(t + 2 < NCHT)
            def _():
                idx_start(t + 2, bp2)
        return 0
    lax.fori_loop(0, NCHT // NBUF, step, 0)
    gather_wait(NCHT - 2, (NCHT - 2) % NBUF)
    scatter_start(NCHT - 2, (NCHT - 2) % NBUF)
    gather_wait(NCHT - 1, (NCHT - 1) % NBUF)
    scatter_start(NCHT - 1, (NCHT - 1) % NBUF)
    for t in range(NCHT - NBUF, NCHT):
        scatter_wait(t, t % NBUF)
    plsc.subcore_barrier()

    pltpu.sync_copy(acc_sh.at[pl.ds(base_r, RT)],
                    spart.at[c].at[pl.ds(base_r, RT)])


BLK = 1000
NBLK = N // BLK


def _layer_a_body(c_ref, w0_ref, b0_ref, w1_ref, dinv_ref, o_ref):
    x1 = jnp.maximum(c_ref[...] * w0_ref[...] + b0_ref[...], 0.0)
    o_ref[...] = dinv_ref[...] * jnp.dot(
        x1, w1_ref[...], preferred_element_type=jnp.float32)


def _layer_a_call(c2d, W0, b0, W1, dinv2d):
    return pl.pallas_call(
        _layer_a_body,
        grid=(NBLK,),
        in_specs=[
            pl.BlockSpec((BLK, 1), lambda i: (i, 0)),
            pl.BlockSpec((1, H), lambda i: (0, 0)),
            pl.BlockSpec((1, H), lambda i: (0, 0)),
            pl.BlockSpec((H, H), lambda i: (0, 0)),
            pl.BlockSpec((BLK, 1), lambda i: (i, 0)),
        ],
        out_specs=pl.BlockSpec((BLK, H), lambda i: (i, 0)),
        out_shape=jax.ShapeDtypeStruct((N, H), jnp.float32),
    )(c2d, W0, b0, W1, dinv2d)


def _layer_b_body(sp_ref, y_ref, dinv_ref, b_ref, w_ref, o_ref):
    ssum = sp_ref[0] + sp_ref[1]
    x = jnp.maximum(dinv_ref[...] * (ssum + y_ref[...]) + b_ref[...], 0.0)
    o_ref[...] = dinv_ref[...] * jnp.dot(
        x, w_ref[...], preferred_element_type=jnp.float32)


def _layer_b_call(spart, y, dinv2d, b, W):
    return pl.pallas_call(
        _layer_b_body,
        grid=(NBLK,),
        in_specs=[
            pl.BlockSpec((NC, BLK, H), lambda i: (0, i, 0)),
            pl.BlockSpec((BLK, H), lambda i: (i, 0)),
            pl.BlockSpec((BLK, 1), lambda i: (i, 0)),
            pl.BlockSpec((1, H), lambda i: (0, 0)),
            pl.BlockSpec((H, H), lambda i: (0, 0)),
        ],
        out_specs=pl.BlockSpec((BLK, H), lambda i: (i, 0)),
        out_shape=jax.ShapeDtypeStruct((N, H), jnp.float32),
    )(spart, y, dinv2d, b, W)


def _final_body(sp_ref, y_ref, dinv_ref, b_ref, batch_ref, wp_ref, bp_ref,
                o_ref, acc, cnt):
    i = pl.program_id(0)

    @pl.when(i == 0)
    def _():
        acc[...] = jnp.zeros_like(acc)
        cnt[...] = jnp.zeros_like(cnt)

    ssum = sp_ref[0] + sp_ref[1]
    x = jnp.maximum(dinv_ref[...] * (ssum + y_ref[...]) + b_ref[...], 0.0)
    oh = (batch_ref[...] == lax.broadcasted_iota(jnp.int32, (BLK, G), 1)
          ).astype(jnp.float32)
    acc[...] += lax.dot_general(
        oh, x, (((0,), (0,)), ((), ())), preferred_element_type=jnp.float32)
    cnt[...] += lax.dot_general(
        oh, jnp.ones((BLK, 1), jnp.float32), (((0,), (0,)), ((), ())),
        preferred_element_type=jnp.float32)

    @pl.when(i == pl.num_programs(0) - 1)
    def _():
        pooled = acc[...] / jnp.maximum(cnt[...], 1.0)
        logits = jnp.dot(pooled, wp_ref[...],
                         preferred_element_type=jnp.float32) + bp_ref[...]
        m = jnp.max(logits, axis=1, keepdims=True)
        ex = jnp.exp(logits - m)
        o_ref[...] = logits - m - jnp.log(jnp.sum(ex, axis=1, keepdims=True))


def _final_call(spart, y, dinv2d, b, batch2d, Wp, bp):
    return pl.pallas_call(
        _final_body,
        grid=(NBLK,),
        in_specs=[
            pl.BlockSpec((NC, BLK, H), lambda i: (0, i, 0)),
            pl.BlockSpec((BLK, H), lambda i: (i, 0)),
            pl.BlockSpec((BLK, 1), lambda i: (i, 0)),
            pl.BlockSpec((1, H), lambda i: (0, 0)),
            pl.BlockSpec((BLK, 1), lambda i: (i, 0)),
            pl.BlockSpec((H, NCLS), lambda i: (0, 0)),
            pl.BlockSpec((1, NCLS), lambda i: (0, 0)),
        ],
        out_specs=pl.BlockSpec((G, NCLS), lambda i: (0, 0)),
        out_shape=jax.ShapeDtypeStruct((G, NCLS), jnp.float32),
        scratch_shapes=[
            pltpu.VMEM((G, H), jnp.float32),
            pltpu.VMEM((G, 1), jnp.float32),
        ],
    )(spart, y, dinv2d, b, batch2d, Wp, bp)


def kernel(edge_index, batch, W0, b0, W1, b1, W2, b2, Wp, bp):
    src = edge_index[0]
    dst = edge_index[1]

    dinv_p, c_p = _scalar_stage(src, dst)
    dinv2d = dinv_p[:N, None]
    c2d = c_p[:N, None]

    y1 = _layer_a_call(c2d, W0, b0[None, :], W1, dinv2d)
    s1 = _prop_stage(y1, src, dst)
    y2 = _layer_b_call(s1, y1, dinv2d, b1[None, :], W2)
    s2 = _prop_stage(y2, src, dst)
    return _final_call(s2, y2, dinv2d, b2[None, :], batch[:, None],
                       Wp, bp[None, :])
